# baseline (speedup 1.0000x reference)
import jax
import jax.numpy as jnp
from jax.experimental import pallas as pl

_BS, _H, _W, _NF = 16, 32, 32, 128


def _body(ce_ref, re_ref, out_ref):
    out_ref[0, :, :, 0:_NF] = jnp.broadcast_to(
        ce_ref[...][None, :, :], (_H, _W, _NF)
    )
    out_ref[0, :, :, _NF : 2 * _NF] = jnp.broadcast_to(
        re_ref[...][:, None, :], (_H, _W, _NF)
    )


def kernel(mask, row_embed, col_embed):
    bs, h, w = mask.shape
    out = pl.pallas_call(
        _body,
        grid=(_BS,),
        in_specs=[
            pl.BlockSpec((_W, _NF), lambda b: (0, 0)),
            pl.BlockSpec((_H, _NF), lambda b: (0, 0)),
        ],
        out_specs=pl.BlockSpec((1, _H, _W, 2 * _NF), lambda b: (b, 0, 0, 0)),
        out_shape=jax.ShapeDtypeStruct((_BS, _H, _W, 2 * _NF), jnp.float32),
    )(col_embed, row_embed)
    return jnp.transpose(out, (0, 3, 1, 2))
